# baseline (speedup 1.0000x reference)
import jax
import jax.numpy as jnp
from jax import lax
from jax.experimental import pallas as pl
from jax.experimental.pallas import tpu as pltpu
from jax.experimental.pallas import tpu_sc as plsc

_L = 16
_NW = 32
_VC = 1024
_U = 8


def _gate_body(hs_ref, gw_ref, i1_ref, i2_ref, w1_ref, w2_ref, aux_ref):
    logits = lax.dot_general(
        hs_ref[...], gw_ref[...], (((1,), (1,)), ((), ())),
        preferred_element_type=jnp.float32)
    m = jnp.max(logits, axis=-1, keepdims=True)
    e = jnp.exp(logits - m)
    probs = e / jnp.sum(e, axis=-1, keepdims=True)
    ne = probs.shape[-1]
    eidx = lax.broadcasted_iota(jnp.int32, probs.shape, 1)
    m1 = jnp.max(probs, axis=-1, keepdims=True)
    i1 = jnp.min(jnp.where(probs == m1, eidx, ne), axis=-1, keepdims=True)
    masked = jnp.where(eidx == i1, -jnp.inf, probs)
    m2 = jnp.max(masked, axis=-1, keepdims=True)
    i2 = jnp.min(jnp.where(masked == m2, eidx, ne), axis=-1, keepdims=True)
    denom = m1 + m2
    i1_ref[...] = i1
    i2_ref[...] = i2
    w1_ref[...] = m1 / denom
    w2_ref[...] = m2 / denom
    usage = jnp.mean(probs, axis=0, keepdims=True)
    aux_ref[...] = jnp.sum(usage * jnp.log(usage), axis=-1,
                           keepdims=True) * ne


def _gate(hidden_states, gate_weight):
    t = hidden_states.shape[0]
    return pl.pallas_call(
        _gate_body,
        out_shape=[
            jax.ShapeDtypeStruct((t, 1), jnp.int32),
            jax.ShapeDtypeStruct((t, 1), jnp.int32),
            jax.ShapeDtypeStruct((t, 1), jnp.float32),
            jax.ShapeDtypeStruct((t, 1), jnp.float32),
            jax.ShapeDtypeStruct((1, 1), jnp.float32),
        ],
    )(hidden_states, gate_weight)


def _make_sc(t, e, v):
    ng = t // _L
    last0 = v - _VC
    mesh = plsc.VectorSubcoreMesh(
        core_axis_name="c", subcore_axis_name="s",
        num_cores=2, num_subcores=16)

    def body(eb_hbm, i1_hbm, i2_hbm, w1_hbm, w2_hbm, out_hbm,
             table_v, i1_v, i2_v, w1_v, w2_v, buf0, buf1, sem0, sem1):
        wid = lax.axis_index("s") * 2 + lax.axis_index("c")
        col0 = lax.select(wid < _NW - 1, wid * _VC, jnp.int32(last0))
        col0 = pl.multiple_of(col0, 8)
        pltpu.sync_copy(eb_hbm.at[:, pl.ds(col0, _VC)], table_v)
        pltpu.sync_copy(i1_hbm, i1_v)
        pltpu.sync_copy(i2_hbm, i2_v)
        pltpu.sync_copy(w1_hbm, w1_v)
        pltpu.sync_copy(w2_hbm, w2_v)
        lane = lax.iota(jnp.int32, _L)
        bufs = (buf0, buf1)
        sems = (sem0, sem1)
        pending = [None, None]
        for g in range(ng):
            buf = bufs[g % 2]
            if pending[g % 2] is not None:
                pending[g % 2].wait()
            i1g = i1_v[pl.ds(g * _L, _L)]
            i2g = i2_v[pl.ds(g * _L, _L)]
            w1g = w1_v[pl.ds(g * _L, _L)]
            w2g = w2_v[pl.ds(g * _L, _L)]

            def cbody(cb, carry, buf=buf, i1g=i1g, i2g=i2g,
                      w1g=w1g, w2g=w2g):
                for u in range(_U):
                    c = cb * _U + u
                    cs = jnp.full((_L,), c, jnp.int32)
                    g1 = plsc.load_gather(table_v, [i1g, cs])
                    g2 = plsc.load_gather(table_v, [i2g, cs])
                    plsc.store_scatter(buf, [lane, cs], w1g * g1 + w2g * g2)
                return carry

            lax.fori_loop(0, _VC // _U, cbody, 0)
            pending[g % 2] = pltpu.async_copy(
                buf,
                out_hbm.at[pl.ds(g * _L, _L), pl.ds(col0, _VC)],
                sems[g % 2])
        pending[0].wait()
        pending[1].wait()

    return pl.kernel(
        body,
        out_type=jax.ShapeDtypeStruct((t, v), jnp.float32),
        mesh=mesh,
        compiler_params=pltpu.CompilerParams(
            use_tc_tiling_on_sc=False, needs_layout_passes=False),
        scratch_types=[
            pltpu.VMEM((e, _VC), jnp.float32),
            pltpu.VMEM((t,), jnp.int32),
            pltpu.VMEM((t,), jnp.int32),
            pltpu.VMEM((t,), jnp.float32),
            pltpu.VMEM((t,), jnp.float32),
            pltpu.VMEM((_L, _VC), jnp.float32),
            pltpu.VMEM((_L, _VC), jnp.float32),
            pltpu.SemaphoreType.DMA,
            pltpu.SemaphoreType.DMA,
        ],
    )


def kernel(hidden_states, gate_weight, expert_biases):
    t = hidden_states.shape[0]
    e, v = expert_biases.shape
    i1, i2, w1, w2, aux = _gate(hidden_states, gate_weight)
    sc = _make_sc(t, e, v)
    bias = sc(expert_biases, i1.reshape(t), i2.reshape(t),
              w1.reshape(t), w2.reshape(t))
    return bias, aux[0, 0]


# speedup vs baseline: 2.2722x; 2.2722x over previous
import jax
import jax.numpy as jnp
from jax import lax
from jax.experimental import pallas as pl
from jax.experimental.pallas import tpu as pltpu
from jax.experimental.pallas import tpu_sc as plsc

_L = 16
_NW = 32
_VC = 1024
_U = 8
_VP = _VC + 1


def _gate_body(hs_ref, gw_ref, i1_ref, i2_ref, w1_ref, w2_ref, aux_ref):
    logits = lax.dot_general(
        hs_ref[...], gw_ref[...], (((1,), (1,)), ((), ())),
        preferred_element_type=jnp.float32)
    m = jnp.max(logits, axis=-1, keepdims=True)
    e = jnp.exp(logits - m)
    probs = e / jnp.sum(e, axis=-1, keepdims=True)
    ne = probs.shape[-1]
    eidx = lax.broadcasted_iota(jnp.int32, probs.shape, 1)
    m1 = jnp.max(probs, axis=-1, keepdims=True)
    i1 = jnp.min(jnp.where(probs == m1, eidx, ne), axis=-1, keepdims=True)
    masked = jnp.where(eidx == i1, -jnp.inf, probs)
    m2 = jnp.max(masked, axis=-1, keepdims=True)
    i2 = jnp.min(jnp.where(masked == m2, eidx, ne), axis=-1, keepdims=True)
    denom = m1 + m2
    i1_ref[...] = i1
    i2_ref[...] = i2
    w1_ref[...] = m1 / denom
    w2_ref[...] = m2 / denom
    usage = jnp.mean(probs, axis=0, keepdims=True)
    aux_ref[...] = jnp.sum(usage * jnp.log(usage), axis=-1,
                           keepdims=True) * ne


def _gate(hidden_states, gate_weight):
    t = hidden_states.shape[0]
    return pl.pallas_call(
        _gate_body,
        out_shape=[
            jax.ShapeDtypeStruct((t, 1), jnp.int32),
            jax.ShapeDtypeStruct((t, 1), jnp.int32),
            jax.ShapeDtypeStruct((t, 1), jnp.float32),
            jax.ShapeDtypeStruct((t, 1), jnp.float32),
            jax.ShapeDtypeStruct((1, 1), jnp.float32),
        ],
    )(hidden_states, gate_weight)


def _make_sc(t, e, v):
    ng = t // _L
    last0 = v - _VC
    mesh = plsc.VectorSubcoreMesh(
        core_axis_name="c", subcore_axis_name="s",
        num_cores=2, num_subcores=16)

    def body(eb_hbm, i1_hbm, i2_hbm, w1_hbm, w2_hbm, out_hbm,
             table_v, i1_v, i2_v, w1_v, w2_v, buf0, buf1, sem0, sem1):
        wid = lax.axis_index("s") * 2 + lax.axis_index("c")
        col0 = lax.select(wid < _NW - 1, wid * _VC, jnp.int32(last0))
        col0 = pl.multiple_of(col0, 8)
        pltpu.sync_copy(eb_hbm.at[:, pl.ds(col0, _VC)],
                        table_v.at[:, pl.ds(0, _VC)])
        pltpu.sync_copy(i1_hbm, i1_v)
        pltpu.sync_copy(i2_hbm, i2_v)
        pltpu.sync_copy(w1_hbm, w1_v)
        pltpu.sync_copy(w2_hbm, w2_v)
        lane = lax.iota(jnp.int32, _L)
        bufs = (buf0, buf1)
        sems = (sem0, sem1)
        pending = [None, None]
        for g in range(ng):
            buf = bufs[g % 2]
            if pending[g % 2] is not None:
                pending[g % 2].wait()
            i1g = i1_v[pl.ds(g * _L, _L)]
            i2g = i2_v[pl.ds(g * _L, _L)]
            w1g = w1_v[pl.ds(g * _L, _L)]
            w2g = w2_v[pl.ds(g * _L, _L)]

            def cbody(cb, carry, buf=buf, i1g=i1g, i2g=i2g,
                      w1g=w1g, w2g=w2g):
                for u in range(_U):
                    c = cb * _U + u
                    cs = jnp.full((_L,), c, jnp.int32)
                    g1 = plsc.load_gather(table_v, [i1g, cs])
                    g2 = plsc.load_gather(table_v, [i2g, cs])
                    plsc.store_scatter(buf, [lane, cs], w1g * g1 + w2g * g2)
                return carry

            lax.fori_loop(0, _VC // _U, cbody, 0)
            pending[g % 2] = pltpu.async_copy(
                buf.at[:, pl.ds(0, _VC)],
                out_hbm.at[pl.ds(g * _L, _L), pl.ds(col0, _VC)],
                sems[g % 2])
        pending[0].wait()
        pending[1].wait()

    return pl.kernel(
        body,
        out_type=jax.ShapeDtypeStruct((t, v), jnp.float32),
        mesh=mesh,
        compiler_params=pltpu.CompilerParams(
            use_tc_tiling_on_sc=False, needs_layout_passes=False),
        scratch_types=[
            pltpu.VMEM((e, _VP), jnp.float32),
            pltpu.VMEM((t,), jnp.int32),
            pltpu.VMEM((t,), jnp.int32),
            pltpu.VMEM((t,), jnp.float32),
            pltpu.VMEM((t,), jnp.float32),
            pltpu.VMEM((_L, _VP), jnp.float32),
            pltpu.VMEM((_L, _VP), jnp.float32),
            pltpu.SemaphoreType.DMA,
            pltpu.SemaphoreType.DMA,
        ],
    )


def kernel(hidden_states, gate_weight, expert_biases):
    t = hidden_states.shape[0]
    e, v = expert_biases.shape
    i1, i2, w1, w2, aux = _gate(hidden_states, gate_weight)
    sc = _make_sc(t, e, v)
    bias = sc(expert_biases, i1.reshape(t), i2.reshape(t),
              w1.reshape(t), w2.reshape(t))
    return bias, aux[0, 0]


# speedup vs baseline: 2.2758x; 1.0016x over previous
import jax
import jax.numpy as jnp
from jax import lax
from jax.experimental import pallas as pl
from jax.experimental.pallas import tpu as pltpu
from jax.experimental.pallas import tpu_sc as plsc

_L = 16
_NW = 32
_VC = 1024
_U = 8
_VP = _VC + 8


def _gate_body(hs_ref, gw_ref, i1_ref, i2_ref, w1_ref, w2_ref, aux_ref):
    logits = lax.dot_general(
        hs_ref[...], gw_ref[...], (((1,), (1,)), ((), ())),
        preferred_element_type=jnp.float32)
    m = jnp.max(logits, axis=-1, keepdims=True)
    e = jnp.exp(logits - m)
    probs = e / jnp.sum(e, axis=-1, keepdims=True)
    ne = probs.shape[-1]
    eidx = lax.broadcasted_iota(jnp.int32, probs.shape, 1)
    m1 = jnp.max(probs, axis=-1, keepdims=True)
    i1 = jnp.min(jnp.where(probs == m1, eidx, ne), axis=-1, keepdims=True)
    masked = jnp.where(eidx == i1, -jnp.inf, probs)
    m2 = jnp.max(masked, axis=-1, keepdims=True)
    i2 = jnp.min(jnp.where(masked == m2, eidx, ne), axis=-1, keepdims=True)
    denom = m1 + m2
    i1_ref[...] = i1
    i2_ref[...] = i2
    w1_ref[...] = m1 / denom
    w2_ref[...] = m2 / denom
    usage = jnp.mean(probs, axis=0, keepdims=True)
    aux_ref[...] = jnp.sum(usage * jnp.log(usage), axis=-1,
                           keepdims=True) * ne


def _gate(hidden_states, gate_weight):
    t = hidden_states.shape[0]
    return pl.pallas_call(
        _gate_body,
        out_shape=[
            jax.ShapeDtypeStruct((t, 1), jnp.int32),
            jax.ShapeDtypeStruct((t, 1), jnp.int32),
            jax.ShapeDtypeStruct((t, 1), jnp.float32),
            jax.ShapeDtypeStruct((t, 1), jnp.float32),
            jax.ShapeDtypeStruct((1, 1), jnp.float32),
        ],
    )(hidden_states, gate_weight)


def _make_sc(t, e, v):
    ng = t // _L
    last0 = v - _VC
    mesh = plsc.VectorSubcoreMesh(
        core_axis_name="c", subcore_axis_name="s",
        num_cores=2, num_subcores=16)

    def body(eb_hbm, i1_hbm, i2_hbm, w1_hbm, w2_hbm, out_hbm,
             table_v, i1_v, i2_v, w1_v, w2_v, buf0, buf1, sem0, sem1):
        wid = lax.axis_index("s") * 2 + lax.axis_index("c")
        col0 = lax.select(wid < _NW - 1, wid * _VC, jnp.int32(last0))
        col0 = pl.multiple_of(col0, 8)
        pltpu.sync_copy(eb_hbm.at[:, pl.ds(col0, _VC)],
                        table_v.at[:, pl.ds(0, _VC)])
        pltpu.sync_copy(i1_hbm, i1_v)
        pltpu.sync_copy(i2_hbm, i2_v)
        pltpu.sync_copy(w1_hbm, w1_v)
        pltpu.sync_copy(w2_hbm, w2_v)
        lane = lax.iota(jnp.int32, _L)
        bufs = (buf0, buf1)
        sems = (sem0, sem1)
        pending = [None, None]
        for g in range(ng):
            buf = bufs[g % 2]
            if pending[g % 2] is not None:
                pending[g % 2].wait()
            i1g = i1_v[pl.ds(g * _L, _L)]
            i2g = i2_v[pl.ds(g * _L, _L)]
            w1g = w1_v[pl.ds(g * _L, _L)]
            w2g = w2_v[pl.ds(g * _L, _L)]

            def cbody(cb, carry, buf=buf, i1g=i1g, i2g=i2g,
                      w1g=w1g, w2g=w2g):
                for u in range(_U):
                    c = cb * _U + u
                    cs = jnp.full((_L,), c, jnp.int32)
                    g1 = plsc.load_gather(table_v, [i1g, cs])
                    g2 = plsc.load_gather(table_v, [i2g, cs])
                    plsc.store_scatter(buf, [lane, cs], w1g * g1 + w2g * g2)
                return carry

            lax.fori_loop(0, _VC // _U, cbody, 0)
            pending[g % 2] = pltpu.async_copy(
                buf.at[:, pl.ds(0, _VC)],
                out_hbm.at[pl.ds(g * _L, _L), pl.ds(col0, _VC)],
                sems[g % 2])
        pending[0].wait()
        pending[1].wait()

    return pl.kernel(
        body,
        out_type=jax.ShapeDtypeStruct((t, v), jnp.float32),
        mesh=mesh,
        compiler_params=pltpu.CompilerParams(
            use_tc_tiling_on_sc=False, needs_layout_passes=False),
        scratch_types=[
            pltpu.VMEM((e, _VP), jnp.float32),
            pltpu.VMEM((t,), jnp.int32),
            pltpu.VMEM((t,), jnp.int32),
            pltpu.VMEM((t,), jnp.float32),
            pltpu.VMEM((t,), jnp.float32),
            pltpu.VMEM((_L, _VP), jnp.float32),
            pltpu.VMEM((_L, _VP), jnp.float32),
            pltpu.SemaphoreType.DMA,
            pltpu.SemaphoreType.DMA,
        ],
    )


def kernel(hidden_states, gate_weight, expert_biases):
    t = hidden_states.shape[0]
    e, v = expert_biases.shape
    i1, i2, w1, w2, aux = _gate(hidden_states, gate_weight)
    sc = _make_sc(t, e, v)
    bias = sc(expert_biases, i1.reshape(t), i2.reshape(t),
              w1.reshape(t), w2.reshape(t))
    return bias, aux[0, 0]


# speedup vs baseline: 3.2806x; 1.4415x over previous
import jax
import jax.numpy as jnp
from jax import lax
from jax.experimental import pallas as pl
from jax.experimental.pallas import tpu as pltpu
from jax.experimental.pallas import tpu_sc as plsc

_L = 16
_NW = 32
_VC = 1024
_U = 8
_VP = _VC + 8


def _gate_body(hs_ref, gw_ref, i1_ref, i2_ref, w1_ref, w2_ref, aux_ref):
    logits = lax.dot_general(
        hs_ref[...], gw_ref[...], (((1,), (1,)), ((), ())),
        preferred_element_type=jnp.float32)
    m = jnp.max(logits, axis=-1, keepdims=True)
    e = jnp.exp(logits - m)
    probs = e / jnp.sum(e, axis=-1, keepdims=True)
    ne = probs.shape[-1]
    eidx = lax.broadcasted_iota(jnp.int32, probs.shape, 1)
    m1 = jnp.max(probs, axis=-1, keepdims=True)
    i1 = jnp.min(jnp.where(probs == m1, eidx, ne), axis=-1, keepdims=True)
    masked = jnp.where(eidx == i1, -jnp.inf, probs)
    m2 = jnp.max(masked, axis=-1, keepdims=True)
    i2 = jnp.min(jnp.where(masked == m2, eidx, ne), axis=-1, keepdims=True)
    denom = m1 + m2
    i1_ref[...] = i1
    i2_ref[...] = i2
    w1_ref[...] = m1 / denom
    w2_ref[...] = m2 / denom
    usage = jnp.mean(probs, axis=0, keepdims=True)
    aux_ref[...] = jnp.sum(usage * jnp.log(usage), axis=-1,
                           keepdims=True) * ne


def _gate(hidden_states, gate_weight):
    t = hidden_states.shape[0]
    return pl.pallas_call(
        _gate_body,
        out_shape=[
            jax.ShapeDtypeStruct((t, 1), jnp.int32),
            jax.ShapeDtypeStruct((t, 1), jnp.int32),
            jax.ShapeDtypeStruct((t, 1), jnp.float32),
            jax.ShapeDtypeStruct((t, 1), jnp.float32),
            jax.ShapeDtypeStruct((1, 1), jnp.float32),
        ],
    )(hidden_states, gate_weight)


def _make_sc(t, e, v):
    ng = t // _L
    last0 = v - _VC
    mesh = plsc.VectorSubcoreMesh(
        core_axis_name="c", subcore_axis_name="s",
        num_cores=2, num_subcores=16)

    def body(eb_hbm, i1_hbm, i2_hbm, w1_hbm, w2_hbm, out_hbm,
             table_v, i1_v, i2_v, w1_v, w2_v, buf0, buf1, sem0, sem1):
        wid = lax.axis_index("s") * 2 + lax.axis_index("c")
        col0 = lax.select(wid < _NW - 1, wid * _VC, jnp.int32(last0))
        col0 = pl.multiple_of(col0, 8)
        pltpu.sync_copy(eb_hbm.at[:, pl.ds(col0, _VC)],
                        table_v.at[:, pl.ds(0, _VC)])
        pltpu.sync_copy(i1_hbm, i1_v)
        pltpu.sync_copy(i2_hbm, i2_v)
        pltpu.sync_copy(w1_hbm, w1_v)
        pltpu.sync_copy(w2_hbm, w2_v)
        lane = lax.iota(jnp.int32, _L)
        bufs = (buf0, buf1)
        sems = (sem0, sem1)
        pending = [None, None]
        for g in range(ng):
            buf = bufs[g % 2]
            if pending[g % 2] is not None:
                pending[g % 2].wait()
            i1g = i1_v[pl.ds(g * _L, _L)]
            i2g = i2_v[pl.ds(g * _L, _L)]
            w1g = w1_v[pl.ds(g * _L, _L)]
            w2g = w2_v[pl.ds(g * _L, _L)]

            def tbody(j, carry, buf=buf, i1g=i1g, i2g=i2g,
                      w1g=w1g, w2g=w2g):
                sel = lane == j
                i1s = lax.reduce_max(jnp.where(sel, i1g, 0), axes=(0,))
                i2s = lax.reduce_max(jnp.where(sel, i2g, 0), axes=(0,))
                w1s = lax.reduce_max(jnp.where(sel, w1g, 0.0), axes=(0,))
                w2s = lax.reduce_max(jnp.where(sel, w2g, 0.0), axes=(0,))
                w1b = jnp.full((_L,), w1s)
                w2b = jnp.full((_L,), w2s)

                def cbody(cb, carry2):
                    for u in range(_U):
                        c = (cb * _U + u) * _L
                        v1 = table_v[i1s, pl.ds(c, _L)]
                        v2 = table_v[i2s, pl.ds(c, _L)]
                        buf[j, pl.ds(c, _L)] = w1b * v1 + w2b * v2
                    return carry2

                lax.fori_loop(0, _VC // (_L * _U), cbody, 0)
                return carry

            lax.fori_loop(0, _L, tbody, 0)
            pending[g % 2] = pltpu.async_copy(
                buf.at[:, pl.ds(0, _VC)],
                out_hbm.at[pl.ds(g * _L, _L), pl.ds(col0, _VC)],
                sems[g % 2])
        pending[0].wait()
        pending[1].wait()

    return pl.kernel(
        body,
        out_type=jax.ShapeDtypeStruct((t, v), jnp.float32),
        mesh=mesh,
        compiler_params=pltpu.CompilerParams(
            use_tc_tiling_on_sc=False, needs_layout_passes=False),
        scratch_types=[
            pltpu.VMEM((e, _VP), jnp.float32),
            pltpu.VMEM((t,), jnp.int32),
            pltpu.VMEM((t,), jnp.int32),
            pltpu.VMEM((t,), jnp.float32),
            pltpu.VMEM((t,), jnp.float32),
            pltpu.VMEM((_L, _VP), jnp.float32),
            pltpu.VMEM((_L, _VP), jnp.float32),
            pltpu.SemaphoreType.DMA,
            pltpu.SemaphoreType.DMA,
        ],
    )


def kernel(hidden_states, gate_weight, expert_biases):
    t = hidden_states.shape[0]
    e, v = expert_biases.shape
    i1, i2, w1, w2, aux = _gate(hidden_states, gate_weight)
    sc = _make_sc(t, e, v)
    bias = sc(expert_biases, i1.reshape(t), i2.reshape(t),
              w1.reshape(t), w2.reshape(t))
    return bias, aux[0, 0]


# speedup vs baseline: 7.5684x; 2.3070x over previous
import jax
import jax.numpy as jnp
from jax import lax
from jax.experimental import pallas as pl
from jax.experimental.pallas import tpu as pltpu
from jax.experimental.pallas import tpu_sc as plsc

_L = 16
_NW = 32
_VC = 1024
_U = 8
_VP = _VC + 8


def _gate_body(hs_ref, gw_ref, i1_ref, i2_ref, w1_ref, w2_ref, aux_ref):
    logits = lax.dot_general(
        hs_ref[...], gw_ref[...], (((1,), (1,)), ((), ())),
        preferred_element_type=jnp.float32)
    m = jnp.max(logits, axis=-1, keepdims=True)
    e = jnp.exp(logits - m)
    probs = e / jnp.sum(e, axis=-1, keepdims=True)
    ne = probs.shape[-1]
    eidx = lax.broadcasted_iota(jnp.int32, probs.shape, 1)
    m1 = jnp.max(probs, axis=-1, keepdims=True)
    i1 = jnp.min(jnp.where(probs == m1, eidx, ne), axis=-1, keepdims=True)
    masked = jnp.where(eidx == i1, -jnp.inf, probs)
    m2 = jnp.max(masked, axis=-1, keepdims=True)
    i2 = jnp.min(jnp.where(masked == m2, eidx, ne), axis=-1, keepdims=True)
    denom = m1 + m2
    i1_ref[...] = i1
    i2_ref[...] = i2
    w1_ref[...] = m1 / denom
    w2_ref[...] = m2 / denom
    usage = jnp.mean(probs, axis=0, keepdims=True)
    aux_ref[...] = jnp.sum(usage * jnp.log(usage), axis=-1,
                           keepdims=True) * ne


def _gate(hidden_states, gate_weight):
    t = hidden_states.shape[0]
    return pl.pallas_call(
        _gate_body,
        out_shape=[
            jax.ShapeDtypeStruct((t, 1), jnp.int32),
            jax.ShapeDtypeStruct((t, 1), jnp.int32),
            jax.ShapeDtypeStruct((t, 1), jnp.float32),
            jax.ShapeDtypeStruct((t, 1), jnp.float32),
            jax.ShapeDtypeStruct((1, 1), jnp.float32),
        ],
    )(hidden_states, gate_weight)


def _make_sc(t, e, v):
    ng = t // _L
    last0 = v - _VC
    mesh = plsc.VectorSubcoreMesh(
        core_axis_name="c", subcore_axis_name="s",
        num_cores=2, num_subcores=16)

    def body(eb_hbm, i1_hbm, i2_hbm, w1_hbm, w2_hbm, out_hbm,
             table_v, i1_v, i2_v, w1_v, w2_v, buf0, buf1, sem0, sem1):
        wid = lax.axis_index("s") * 2 + lax.axis_index("c")
        col0 = lax.select(wid < _NW - 1, wid * _VC, jnp.int32(last0))
        col0 = pl.multiple_of(col0, 8)
        pltpu.sync_copy(eb_hbm.at[:, pl.ds(col0, _VC)],
                        table_v.at[:, pl.ds(0, _VC)])
        pltpu.sync_copy(i1_hbm, i1_v)
        pltpu.sync_copy(i2_hbm, i2_v)
        pltpu.sync_copy(w1_hbm, w1_v)
        pltpu.sync_copy(w2_hbm, w2_v)
        lane = lax.iota(jnp.int32, _L)
        bufs = (buf0, buf1)
        sems = (sem0, sem1)
        pending = [None, None]
        for g in range(ng):
            buf = bufs[g % 2]
            if pending[g % 2] is not None:
                pending[g % 2].wait()
            i1g = i1_v[pl.ds(g * _L, _L)]
            i2g = i2_v[pl.ds(g * _L, _L)]
            w1g = w1_v[pl.ds(g * _L, _L)]
            w2g = w2_v[pl.ds(g * _L, _L)]

            def tbody(j, carry, buf=buf, i1g=i1g, i2g=i2g,
                      w1g=w1g, w2g=w2g):
                sel = lane == j
                i1s = lax.reduce_max(jnp.where(sel, i1g, 0), axes=(0,))
                i2s = lax.reduce_max(jnp.where(sel, i2g, 0), axes=(0,))
                w1s = lax.reduce_max(jnp.where(sel, w1g, 0.0), axes=(0,))
                w2s = lax.reduce_max(jnp.where(sel, w2g, 0.0), axes=(0,))
                w1b = jnp.full((_L,), w1s)
                w2b = jnp.full((_L,), w2s)

                @plsc.parallel_loop(0, _VC // _L, unroll=_U)
                def cbody(cb):
                    c = cb * _L
                    v1 = table_v[i1s, pl.ds(c, _L)]
                    v2 = table_v[i2s, pl.ds(c, _L)]
                    buf[j, pl.ds(c, _L)] = w1b * v1 + w2b * v2

                return carry

            lax.fori_loop(0, _L, tbody, 0)
            pending[g % 2] = pltpu.async_copy(
                buf.at[:, pl.ds(0, _VC)],
                out_hbm.at[pl.ds(g * _L, _L), pl.ds(col0, _VC)],
                sems[g % 2])
        pending[0].wait()
        pending[1].wait()

    return pl.kernel(
        body,
        out_type=jax.ShapeDtypeStruct((t, v), jnp.float32),
        mesh=mesh,
        compiler_params=pltpu.CompilerParams(
            use_tc_tiling_on_sc=False, needs_layout_passes=False),
        scratch_types=[
            pltpu.VMEM((e, _VP), jnp.float32),
            pltpu.VMEM((t,), jnp.int32),
            pltpu.VMEM((t,), jnp.int32),
            pltpu.VMEM((t,), jnp.float32),
            pltpu.VMEM((t,), jnp.float32),
            pltpu.VMEM((_L, _VP), jnp.float32),
            pltpu.VMEM((_L, _VP), jnp.float32),
            pltpu.SemaphoreType.DMA,
            pltpu.SemaphoreType.DMA,
        ],
    )


def kernel(hidden_states, gate_weight, expert_biases):
    t = hidden_states.shape[0]
    e, v = expert_biases.shape
    i1, i2, w1, w2, aux = _gate(hidden_states, gate_weight)
    sc = _make_sc(t, e, v)
    bias = sc(expert_biases, i1.reshape(t), i2.reshape(t),
              w1.reshape(t), w2.reshape(t))
    return bias, aux[0, 0]


# speedup vs baseline: 7.6761x; 1.0142x over previous
import jax
import jax.numpy as jnp
from jax import lax
from jax.experimental import pallas as pl
from jax.experimental.pallas import tpu as pltpu
from jax.experimental.pallas import tpu_sc as plsc

_L = 16
_NW = 32
_VC = 1024
_U = 8
_VP = _VC + 8


def _gate_body(hs_ref, gw_ref, i1_ref, i2_ref, w1_ref, w2_ref, aux_ref):
    logits = lax.dot_general(
        hs_ref[...], gw_ref[...], (((1,), (1,)), ((), ())),
        preferred_element_type=jnp.float32)
    m = jnp.max(logits, axis=-1, keepdims=True)
    e = jnp.exp(logits - m)
    probs = e / jnp.sum(e, axis=-1, keepdims=True)
    ne = probs.shape[-1]
    eidx = lax.broadcasted_iota(jnp.int32, probs.shape, 1)
    m1 = jnp.max(probs, axis=-1, keepdims=True)
    i1 = jnp.min(jnp.where(probs == m1, eidx, ne), axis=-1, keepdims=True)
    masked = jnp.where(eidx == i1, -jnp.inf, probs)
    m2 = jnp.max(masked, axis=-1, keepdims=True)
    i2 = jnp.min(jnp.where(masked == m2, eidx, ne), axis=-1, keepdims=True)
    denom = m1 + m2
    i1_ref[...] = i1
    i2_ref[...] = i2
    w1_ref[...] = m1 / denom
    w2_ref[...] = m2 / denom
    usage = jnp.mean(probs, axis=0, keepdims=True)
    aux_ref[...] = jnp.sum(usage * jnp.log(usage), axis=-1,
                           keepdims=True) * ne


def _gate(hidden_states, gate_weight):
    t = hidden_states.shape[0]
    return pl.pallas_call(
        _gate_body,
        out_shape=[
            jax.ShapeDtypeStruct((t, 1), jnp.int32),
            jax.ShapeDtypeStruct((t, 1), jnp.int32),
            jax.ShapeDtypeStruct((t, 1), jnp.float32),
            jax.ShapeDtypeStruct((t, 1), jnp.float32),
            jax.ShapeDtypeStruct((1, 1), jnp.float32),
        ],
    )(hidden_states, gate_weight)


def _make_sc(t, e, v):
    ng = t // _L
    last0 = v - _VC
    mesh = plsc.VectorSubcoreMesh(
        core_axis_name="c", subcore_axis_name="s",
        num_cores=2, num_subcores=16)

    def body(eb_hbm, i1_hbm, i2_hbm, w1_hbm, w2_hbm, out_hbm,
             table_v, i1_v, i2_v, w1_v, w2_v, buf0, buf1, sem0, sem1):
        wid = lax.axis_index("s") * 2 + lax.axis_index("c")
        col0 = lax.select(wid < _NW - 1, wid * _VC, jnp.int32(last0))
        col0 = pl.multiple_of(col0, 8)
        pltpu.sync_copy(eb_hbm.at[:, pl.ds(col0, _VC)],
                        table_v.at[:, pl.ds(0, _VC)])
        pltpu.sync_copy(i1_hbm, i1_v.at[pl.ds(0, t)])
        pltpu.sync_copy(i2_hbm, i2_v.at[pl.ds(0, t)])
        pltpu.sync_copy(w1_hbm, w1_v.at[pl.ds(0, t)])
        pltpu.sync_copy(w2_hbm, w2_v.at[pl.ds(0, t)])
        bufs = (buf0, buf1)
        sems = (sem0, sem1)
        pending = [None, None]
        for g in range(ng):
            buf = bufs[g % 2]
            if pending[g % 2] is not None:
                pending[g % 2].wait()

            def tbody(j, carry, buf=buf, g=g):
                tok = g * _L + j
                i1s = i1_v[pl.ds(tok, _L)][0]
                i2s = i2_v[pl.ds(tok, _L)][0]
                w1b = jnp.full((_L,), w1_v[pl.ds(tok, _L)][0])
                w2b = jnp.full((_L,), w2_v[pl.ds(tok, _L)][0])

                @plsc.parallel_loop(0, _VC // _L, unroll=_U)
                def cbody(cb):
                    c = cb * _L
                    v1 = table_v[i1s, pl.ds(c, _L)]
                    v2 = table_v[i2s, pl.ds(c, _L)]
                    buf[j, pl.ds(c, _L)] = w1b * v1 + w2b * v2

                return carry

            lax.fori_loop(0, _L, tbody, 0)
            pending[g % 2] = pltpu.async_copy(
                buf.at[:, pl.ds(0, _VC)],
                out_hbm.at[pl.ds(g * _L, _L), pl.ds(col0, _VC)],
                sems[g % 2])
        pending[0].wait()
        pending[1].wait()

    return pl.kernel(
        body,
        out_type=jax.ShapeDtypeStruct((t, v), jnp.float32),
        mesh=mesh,
        compiler_params=pltpu.CompilerParams(
            use_tc_tiling_on_sc=False, needs_layout_passes=False),
        scratch_types=[
            pltpu.VMEM((e, _VP), jnp.float32),
            pltpu.VMEM((t + _L,), jnp.int32),
            pltpu.VMEM((t + _L,), jnp.int32),
            pltpu.VMEM((t + _L,), jnp.float32),
            pltpu.VMEM((t + _L,), jnp.float32),
            pltpu.VMEM((_L, _VP), jnp.float32),
            pltpu.VMEM((_L, _VP), jnp.float32),
            pltpu.SemaphoreType.DMA,
            pltpu.SemaphoreType.DMA,
        ],
    )


def kernel(hidden_states, gate_weight, expert_biases):
    t = hidden_states.shape[0]
    e, v = expert_biases.shape
    i1, i2, w1, w2, aux = _gate(hidden_states, gate_weight)
    sc = _make_sc(t, e, v)
    bias = sc(expert_biases, i1.reshape(t), i2.reshape(t),
              w1.reshape(t), w2.reshape(t))
    return bias, aux[0, 0]
